# baseline (speedup 1.0000x reference)
import jax
import jax.numpy as jnp
from jax.experimental import pallas as pl
from jax.experimental.pallas import tpu as pltpu

_N = 33554432
_ROWS = 4096
_COLS = 8192
_BLOCK_ROWS = 512


def _body(x_ref, o_ref):
    i = pl.program_id(0)
    x = x_ref[...]
    cond = x <= -1.0
    safe = jnp.minimum(x, -1.0)
    loss = jnp.where(cond, -jnp.log(-safe), x + 1.0)
    s = jnp.sum(loss)

    @pl.when(i == 0)
    def _():
        o_ref[0, 0] = 0.0

    o_ref[0, 0] += s

    @pl.when(i == pl.num_programs(0) - 1)
    def _():
        o_ref[0, 0] = o_ref[0, 0] / _N


def kernel(fx):
    x2d = fx.reshape(_ROWS, _COLS)
    out = pl.pallas_call(
        _body,
        grid=(_ROWS // _BLOCK_ROWS,),
        in_specs=[pl.BlockSpec((_BLOCK_ROWS, _COLS), lambda i: (i, 0))],
        out_specs=pl.BlockSpec(memory_space=pltpu.SMEM),
        out_shape=jax.ShapeDtypeStruct((1, 1), jnp.float32),
        compiler_params=pltpu.CompilerParams(
            dimension_semantics=("arbitrary",),
        ),
    )(x2d)
    return out[0, 0]
